# Initial kernel scaffold; baseline (speedup 1.0000x reference)
import jax
import jax.numpy as jnp
from jax.experimental import pallas as pl


def kernel(x, edge_index, batch, W1a, b1a, W1b, b1b, g1, be1, W2a, b2a, W2b, b2b, g2, be2, W3a, b3a, W3b, b3b, g3, be3, Wc1, bc1, Wc2, bc2):
    raise NotImplementedError("write your pallas kernel here")


# speedup vs baseline: 1.4887x; 1.4887x over previous
import functools

import jax
import jax.numpy as jnp
from jax import lax
from jax.experimental import pallas as pl
from jax.experimental.pallas import tpu as pltpu
from jax.experimental.pallas import tpu_sc as plsc

EPS = 1e-5
NC_SC = 2
NS_SC = 16
NW = NC_SC * NS_SC
CHUNK = 128


def _sc_gather_add(u, v, row, col):
    E = row.shape[0]
    F = u.shape[1]
    n_chunks = E // CHUNK
    mesh = plsc.VectorSubcoreMesh(
        core_axis_name="c", subcore_axis_name="s",
        num_cores=NC_SC, num_subcores=NS_SC)

    @functools.partial(
        pl.kernel,
        out_type=jax.ShapeDtypeStruct((E, F), jnp.float32),
        mesh=mesh,
        scratch_types=[
            pltpu.VMEM((CHUNK,), jnp.int32),
            pltpu.VMEM((CHUNK,), jnp.int32),
            pltpu.VMEM((CHUNK, F), jnp.float32),
            pltpu.VMEM((CHUNK, F), jnp.float32),
            pltpu.SemaphoreType.DMA,
        ],
        compiler_params=pltpu.CompilerParams(use_tc_tiling_on_sc=False),
    )
    def k(u_hbm, v_hbm, row_hbm, col_hbm, a_hbm, idx_r, idx_c, buf_u, buf_v, sem):
        wid = lax.axis_index("s") * NC_SC + lax.axis_index("c")
        my_chunks = (n_chunks - wid + NW - 1) // NW

        def chunk_body(i, carry):
            base = (wid + i * NW) * CHUNK
            pltpu.sync_copy(row_hbm.at[pl.ds(base, CHUNK)], idx_r)
            pltpu.sync_copy(col_hbm.at[pl.ds(base, CHUNK)], idx_c)
            cu = pltpu.async_copy(u_hbm.at[idx_r], buf_u, sem)
            cv = pltpu.async_copy(v_hbm.at[idx_c], buf_v, sem)
            cu.wait()
            cv.wait()

            def add_row(e, c2):
                for j in range(F // 16):
                    sl = pl.ds(j * 16, 16)
                    buf_u[e, sl] = buf_u[e, sl] + buf_v[e, sl]
                return c2

            lax.fori_loop(0, CHUNK, add_row, 0)
            pltpu.sync_copy(buf_u, a_hbm.at[pl.ds(base, CHUNK)])
            return carry

        lax.fori_loop(0, my_chunks, chunk_body, 0)

    return k(u, v, row, col)


def _uv_first(x, Wd, Wr, tn):
    n, f_in = x.shape
    f_out = Wd.shape[0]

    def body(x_ref, wd_ref, wr_ref, u_ref, v_ref):
        xb = x_ref[...]
        dn = (((1,), (1,)), ((), ()))
        u_ref[...] = lax.dot_general(xb, wd_ref[...], dn,
                                     preferred_element_type=jnp.float32, precision=lax.Precision.HIGHEST)
        v_ref[...] = lax.dot_general(xb, wr_ref[...], dn,
                                     preferred_element_type=jnp.float32, precision=lax.Precision.HIGHEST)

    return pl.pallas_call(
        body,
        grid=(n // tn,),
        in_specs=[
            pl.BlockSpec((tn, f_in), lambda i: (i, 0)),
            pl.BlockSpec((f_out, f_in), lambda i: (0, 0)),
            pl.BlockSpec((f_out, f_in), lambda i: (0, 0)),
        ],
        out_specs=[
            pl.BlockSpec((tn, f_out), lambda i: (i, 0)),
            pl.BlockSpec((tn, f_out), lambda i: (i, 0)),
        ],
        out_shape=[
            jax.ShapeDtypeStruct((n, f_out), jnp.float32),
            jax.ShapeDtypeStruct((n, f_out), jnp.float32),
        ],
    )(x, Wd, Wr)


def _norm_uv(hp, s, ss, g, be, Wd, Wr, e_tot, tn):
    n, f_in = hp.shape
    f_out = Wd.shape[0]
    grid = (n // tn,)

    def body(hp_ref, s_ref, ss_ref, g_ref, be_ref, wd_ref, wr_ref,
             u_ref, v_ref):
        mu = s_ref[...] / e_tot
        var = ss_ref[...] / e_tot - mu * mu
        scale = g_ref[...] * lax.rsqrt(var + EPS)
        shift = be_ref[...] - mu * scale
        h = jnp.maximum(hp_ref[...] * scale + shift, 0.0)
        dn = (((1,), (1,)), ((), ()))
        u_ref[...] = lax.dot_general(h, wd_ref[...], dn,
                                     preferred_element_type=jnp.float32, precision=lax.Precision.HIGHEST)
        v_ref[...] = lax.dot_general(h, wr_ref[...], dn,
                                     preferred_element_type=jnp.float32, precision=lax.Precision.HIGHEST)

    vec = lambda f: pl.BlockSpec((1, f), lambda i: (0, 0))
    return pl.pallas_call(
        body,
        grid=grid,
        in_specs=[
            pl.BlockSpec((tn, f_in), lambda i: (i, 0)),
            vec(f_in), vec(f_in), vec(f_in), vec(f_in),
            pl.BlockSpec((f_out, f_in), lambda i: (0, 0)),
            pl.BlockSpec((f_out, f_in), lambda i: (0, 0)),
        ],
        out_specs=[
            pl.BlockSpec((tn, f_out), lambda i: (i, 0)),
            pl.BlockSpec((tn, f_out), lambda i: (i, 0)),
        ],
        out_shape=[
            jax.ShapeDtypeStruct((n, f_out), jnp.float32),
            jax.ShapeDtypeStruct((n, f_out), jnp.float32),
        ],
    )(hp, s, ss, g, be, Wd, Wr)


def _edge_mlp_stats(a, ba, Wb, bb, n_keep, te):
    e, f1 = a.shape
    f2 = Wb.shape[0]
    grid = (e // te,)
    keep_tiles = n_keep // te

    def body(a_ref, ba_ref, wb_ref, bb_ref, hp_ref, s_ref, ss_ref):
        i = pl.program_id(0)
        h = jnp.maximum(a_ref[...] + ba_ref[...], 0.0)
        hp = lax.dot_general(h, wb_ref[...], (((1,), (1,)), ((), ())),
                             preferred_element_type=jnp.float32, precision=lax.Precision.HIGHEST) + bb_ref[...]

        @pl.when(i < keep_tiles)
        def _():
            hp_ref[...] = hp

        @pl.when(i == 0)
        def _():
            s_ref[...] = jnp.zeros_like(s_ref)
            ss_ref[...] = jnp.zeros_like(ss_ref)

        s_ref[...] += hp.sum(axis=0, keepdims=True)
        ss_ref[...] += (hp * hp).sum(axis=0, keepdims=True)

    return pl.pallas_call(
        body,
        grid=grid,
        in_specs=[
            pl.BlockSpec((te, f1), lambda i: (i, 0)),
            pl.BlockSpec((1, f1), lambda i: (0, 0)),
            pl.BlockSpec((f2, f1), lambda i: (0, 0)),
            pl.BlockSpec((1, f2), lambda i: (0, 0)),
        ],
        out_specs=[
            pl.BlockSpec((te, f2), lambda i: (jnp.minimum(i, keep_tiles - 1), 0)),
            pl.BlockSpec((1, f2), lambda i: (0, 0)),
            pl.BlockSpec((1, f2), lambda i: (0, 0)),
        ],
        out_shape=[
            jax.ShapeDtypeStruct((n_keep, f2), jnp.float32),
            jax.ShapeDtypeStruct((1, f2), jnp.float32),
            jax.ShapeDtypeStruct((1, f2), jnp.float32),
        ],
    )(a, ba, Wb, bb)


def _edge_mlp_pool(a, ba, Wb, bb, batch3, n_graphs, te):
    e, f1 = a.shape
    f2 = Wb.shape[0]
    grid = (e // te,)
    neg = -3.4e38
    pos = 3.4e38

    def body(a_ref, ba_ref, wb_ref, bb_ref, bt_ref,
             s_ref, ss_ref, mx_ref, mn_ref):
        i = pl.program_id(0)
        h = jnp.maximum(a_ref[...] + ba_ref[...], 0.0)
        hp = lax.dot_general(h, wb_ref[...], (((1,), (1,)), ((), ())),
                             preferred_element_type=jnp.float32, precision=lax.Precision.HIGHEST) + bb_ref[...]

        @pl.when(i == 0)
        def _():
            s_ref[...] = jnp.zeros_like(s_ref)
            ss_ref[...] = jnp.zeros_like(ss_ref)
            mx_ref[...] = jnp.full_like(mx_ref, neg)
            mn_ref[...] = jnp.full_like(mn_ref, pos)

        s_ref[...] += hp.sum(axis=0, keepdims=True)
        ss_ref[...] += (hp * hp).sum(axis=0, keepdims=True)

        bt = bt_ref[0]
        for g in range(n_graphs):
            m = bt == g
            mx_ref[g, :] = jnp.maximum(
                mx_ref[g, :], jnp.max(jnp.where(m, hp, neg), axis=0))
            mn_ref[g, :] = jnp.minimum(
                mn_ref[g, :], jnp.min(jnp.where(m, hp, pos), axis=0))

    return pl.pallas_call(
        body,
        grid=grid,
        in_specs=[
            pl.BlockSpec((te, f1), lambda i: (i, 0)),
            pl.BlockSpec((1, f1), lambda i: (0, 0)),
            pl.BlockSpec((f2, f1), lambda i: (0, 0)),
            pl.BlockSpec((1, f2), lambda i: (0, 0)),
            pl.BlockSpec((1, te, 1), lambda i: (i, 0, 0)),
        ],
        out_specs=[
            pl.BlockSpec((1, f2), lambda i: (0, 0)),
            pl.BlockSpec((1, f2), lambda i: (0, 0)),
            pl.BlockSpec((n_graphs, f2), lambda i: (0, 0)),
            pl.BlockSpec((n_graphs, f2), lambda i: (0, 0)),
        ],
        out_shape=[
            jax.ShapeDtypeStruct((1, f2), jnp.float32),
            jax.ShapeDtypeStruct((1, f2), jnp.float32),
            jax.ShapeDtypeStruct((n_graphs, f2), jnp.float32),
            jax.ShapeDtypeStruct((n_graphs, f2), jnp.float32),
        ],
    )(a, ba, Wb, bb, batch3)


def _finalize(s, ss, mx, mn, g, be, Wc1, bc1, Wc2, bc2, e_tot):
    n_graphs, f = mx.shape
    nc = Wc2.shape[0]
    fh = Wc1.shape[0]

    def body(s_ref, ss_ref, mx_ref, mn_ref, g_ref, be_ref,
             w1_ref, b1_ref, w2_ref, b2_ref, out_ref):
        mu = s_ref[...] / e_tot
        var = ss_ref[...] / e_tot - mu * mu
        gv = g_ref[...]
        scale = gv * lax.rsqrt(var + EPS)
        shift = be_ref[...] - mu * scale
        pooled_pre = jnp.where(gv >= 0.0, mx_ref[...], mn_ref[...])
        pooled = jnp.maximum(pooled_pre * scale + shift, 0.0)
        dn = (((1,), (1,)), ((), ()))
        z = jnp.maximum(
            lax.dot_general(pooled, w1_ref[...], dn,
                            preferred_element_type=jnp.float32, precision=lax.Precision.HIGHEST) + b1_ref[...],
            0.0)
        out_ref[...] = lax.dot_general(z, w2_ref[...], dn,
                                       preferred_element_type=jnp.float32, precision=lax.Precision.HIGHEST) \
            + b2_ref[...]

    return pl.pallas_call(
        body,
        out_shape=jax.ShapeDtypeStruct((n_graphs, nc), jnp.float32),
    )(s, ss, mx, mn, g, be, Wc1, bc1, Wc2, bc2)


def kernel(x, edge_index, batch,
           W1a, b1a, W1b, b1b, g1, be1,
           W2a, b2a, W2b, b2b, g2, be2,
           W3a, b3a, W3b, b3b, g3, be3,
           Wc1, bc1, Wc2, bc2):
    n = x.shape[0]
    e = edge_index.shape[1]
    e_tot = float(e)
    row = edge_index[0]
    col = edge_index[1]

    def split(Wa):
        f = Wa.shape[1] // 2
        return Wa[:, :f] - Wa[:, f:], Wa[:, f:]

    Wd1, Wr1 = split(W1a)
    Wd2, Wr2 = split(W2a)
    Wd3, Wr3 = split(W3a)
    r2 = lambda t: t.reshape(1, -1)

    u1, v1 = _uv_first(x, Wd1, Wr1, tn=2000)
    a1 = _sc_gather_add(u1, v1, row, col)
    hp1, s1, ss1 = _edge_mlp_stats(a1, r2(b1a), W1b, r2(b1b), n_keep=n, te=2000)

    u2, v2 = _norm_uv(hp1, s1, ss1, r2(g1), r2(be1), Wd2, Wr2, e_tot, tn=2000)
    a2 = _sc_gather_add(u2, v2, row, col)
    hp2, s2, ss2 = _edge_mlp_stats(a2, r2(b2a), W2b, r2(b2b), n_keep=n, te=2000)

    u3, v3 = _norm_uv(hp2, s2, ss2, r2(g2), r2(be2), Wd3, Wr3, e_tot, tn=2000)
    a3 = _sc_gather_add(u3, v3, row, col)
    te3 = 2000
    batch3 = batch.reshape(e // te3, te3, 1)
    s3, ss3, mx, mn = _edge_mlp_pool(a3, r2(b3a), W3b, r2(b3b), batch3,
                                     n_graphs=8, te=te3)

    return _finalize(s3, ss3, mx, mn, r2(g3), r2(be3),
                     Wc1, r2(bc1), Wc2, r2(bc2), e_tot)


# speedup vs baseline: 1.6878x; 1.1338x over previous
import functools

import jax
import jax.numpy as jnp
from jax import lax
from jax.experimental import pallas as pl
from jax.experimental.pallas import tpu as pltpu
from jax.experimental.pallas import tpu_sc as plsc

EPS = 1e-5
NC_SC = 2
NS_SC = 16
NW = NC_SC * NS_SC
CHUNK = 128


def _sc_gather_add(u, v, row, col):
    E = row.shape[0]
    F = u.shape[1]
    n_chunks = E // CHUNK
    mesh = plsc.VectorSubcoreMesh(
        core_axis_name="c", subcore_axis_name="s",
        num_cores=NC_SC, num_subcores=NS_SC)

    @functools.partial(
        pl.kernel,
        out_type=jax.ShapeDtypeStruct((E, F), jnp.float32),
        mesh=mesh,
        scratch_types=[
            pltpu.VMEM((CHUNK,), jnp.int32),
            pltpu.VMEM((CHUNK,), jnp.int32),
            pltpu.VMEM((CHUNK, F), jnp.float32),
            pltpu.VMEM((CHUNK, F), jnp.float32),
            pltpu.SemaphoreType.DMA,
        ],
        compiler_params=pltpu.CompilerParams(use_tc_tiling_on_sc=(F % 128 == 0)),
    )
    def k(u_hbm, v_hbm, row_hbm, col_hbm, a_hbm, idx_r, idx_c, buf_u, buf_v, sem):
        wid = lax.axis_index("s") * NC_SC + lax.axis_index("c")
        my_chunks = (n_chunks - wid + NW - 1) // NW

        def chunk_body(i, carry):
            base = (wid + i * NW) * CHUNK
            pltpu.sync_copy(row_hbm.at[pl.ds(base, CHUNK)], idx_r)
            pltpu.sync_copy(col_hbm.at[pl.ds(base, CHUNK)], idx_c)
            cu = pltpu.async_copy(u_hbm.at[idx_r], buf_u, sem)
            cv = pltpu.async_copy(v_hbm.at[idx_c], buf_v, sem)
            cu.wait()
            cv.wait()

            def add_row(e, c2):
                for j in range(F // 16):
                    sl = pl.ds(j * 16, 16)
                    buf_u[e, sl] = buf_u[e, sl] + buf_v[e, sl]
                return c2

            lax.fori_loop(0, CHUNK, add_row, 0)
            pltpu.sync_copy(buf_u, a_hbm.at[pl.ds(base, CHUNK)])
            return carry

        lax.fori_loop(0, my_chunks, chunk_body, 0)

    return k(u, v, row, col)


def _uv_first(x, Wd, Wr, tn):
    n, f_in = x.shape
    f_out = Wd.shape[0]

    def body(x_ref, wd_ref, wr_ref, u_ref, v_ref):
        xb = x_ref[...]
        dn = (((1,), (1,)), ((), ()))
        u_ref[...] = lax.dot_general(xb, wd_ref[...], dn,
                                     preferred_element_type=jnp.float32, precision=lax.Precision.HIGHEST)
        v_ref[...] = lax.dot_general(xb, wr_ref[...], dn,
                                     preferred_element_type=jnp.float32, precision=lax.Precision.HIGHEST)

    return pl.pallas_call(
        body,
        grid=(n // tn,),
        in_specs=[
            pl.BlockSpec((tn, f_in), lambda i: (i, 0)),
            pl.BlockSpec((f_out, f_in), lambda i: (0, 0)),
            pl.BlockSpec((f_out, f_in), lambda i: (0, 0)),
        ],
        out_specs=[
            pl.BlockSpec((tn, f_out), lambda i: (i, 0)),
            pl.BlockSpec((tn, f_out), lambda i: (i, 0)),
        ],
        out_shape=[
            jax.ShapeDtypeStruct((n, f_out), jnp.float32),
            jax.ShapeDtypeStruct((n, f_out), jnp.float32),
        ],
    )(x, Wd, Wr)


def _norm_uv(hp, s, ss, g, be, Wd, Wr, e_tot, tn):
    n, f_in = hp.shape
    f_out = Wd.shape[0]
    grid = (n // tn,)

    def body(hp_ref, s_ref, ss_ref, g_ref, be_ref, wd_ref, wr_ref,
             u_ref, v_ref):
        mu = s_ref[...] / e_tot
        var = ss_ref[...] / e_tot - mu * mu
        scale = g_ref[...] * lax.rsqrt(var + EPS)
        shift = be_ref[...] - mu * scale
        h = jnp.maximum(hp_ref[...] * scale + shift, 0.0)
        dn = (((1,), (1,)), ((), ()))
        u_ref[...] = lax.dot_general(h, wd_ref[...], dn,
                                     preferred_element_type=jnp.float32, precision=lax.Precision.HIGHEST)
        v_ref[...] = lax.dot_general(h, wr_ref[...], dn,
                                     preferred_element_type=jnp.float32, precision=lax.Precision.HIGHEST)

    vec = lambda f: pl.BlockSpec((1, f), lambda i: (0, 0))
    return pl.pallas_call(
        body,
        grid=grid,
        in_specs=[
            pl.BlockSpec((tn, f_in), lambda i: (i, 0)),
            vec(f_in), vec(f_in), vec(f_in), vec(f_in),
            pl.BlockSpec((f_out, f_in), lambda i: (0, 0)),
            pl.BlockSpec((f_out, f_in), lambda i: (0, 0)),
        ],
        out_specs=[
            pl.BlockSpec((tn, f_out), lambda i: (i, 0)),
            pl.BlockSpec((tn, f_out), lambda i: (i, 0)),
        ],
        out_shape=[
            jax.ShapeDtypeStruct((n, f_out), jnp.float32),
            jax.ShapeDtypeStruct((n, f_out), jnp.float32),
        ],
    )(hp, s, ss, g, be, Wd, Wr)


def _edge_mlp_stats(a, ba, Wb, bb, n_keep, te):
    e, f1 = a.shape
    f2 = Wb.shape[0]
    grid = (e // te,)
    keep_tiles = n_keep // te

    def body(a_ref, ba_ref, wb_ref, bb_ref, hp_ref, s_ref, ss_ref):
        i = pl.program_id(0)
        h = jnp.maximum(a_ref[...] + ba_ref[...], 0.0)
        hp = lax.dot_general(h, wb_ref[...], (((1,), (1,)), ((), ())),
                             preferred_element_type=jnp.float32, precision=lax.Precision.HIGHEST) + bb_ref[...]

        @pl.when(i < keep_tiles)
        def _():
            hp_ref[...] = hp

        @pl.when(i == 0)
        def _():
            s_ref[...] = jnp.zeros_like(s_ref)
            ss_ref[...] = jnp.zeros_like(ss_ref)

        s_ref[...] += hp.sum(axis=0, keepdims=True)
        ss_ref[...] += (hp * hp).sum(axis=0, keepdims=True)

    return pl.pallas_call(
        body,
        grid=grid,
        in_specs=[
            pl.BlockSpec((te, f1), lambda i: (i, 0)),
            pl.BlockSpec((1, f1), lambda i: (0, 0)),
            pl.BlockSpec((f2, f1), lambda i: (0, 0)),
            pl.BlockSpec((1, f2), lambda i: (0, 0)),
        ],
        out_specs=[
            pl.BlockSpec((te, f2), lambda i: (jnp.minimum(i, keep_tiles - 1), 0)),
            pl.BlockSpec((1, f2), lambda i: (0, 0)),
            pl.BlockSpec((1, f2), lambda i: (0, 0)),
        ],
        out_shape=[
            jax.ShapeDtypeStruct((n_keep, f2), jnp.float32),
            jax.ShapeDtypeStruct((1, f2), jnp.float32),
            jax.ShapeDtypeStruct((1, f2), jnp.float32),
        ],
    )(a, ba, Wb, bb)


def _edge_mlp_pool(a, ba, Wb, bb, batch3, n_graphs, te):
    e, f1 = a.shape
    f2 = Wb.shape[0]
    grid = (e // te,)
    neg = -3.4e38
    pos = 3.4e38

    def body(a_ref, ba_ref, wb_ref, bb_ref, bt_ref,
             s_ref, ss_ref, mx_ref, mn_ref):
        i = pl.program_id(0)
        h = jnp.maximum(a_ref[...] + ba_ref[...], 0.0)
        hp = lax.dot_general(h, wb_ref[...], (((1,), (1,)), ((), ())),
                             preferred_element_type=jnp.float32, precision=lax.Precision.HIGHEST) + bb_ref[...]

        @pl.when(i == 0)
        def _():
            s_ref[...] = jnp.zeros_like(s_ref)
            ss_ref[...] = jnp.zeros_like(ss_ref)
            mx_ref[...] = jnp.full_like(mx_ref, neg)
            mn_ref[...] = jnp.full_like(mn_ref, pos)

        s_ref[...] += hp.sum(axis=0, keepdims=True)
        ss_ref[...] += (hp * hp).sum(axis=0, keepdims=True)

        bt = bt_ref[0]
        for g in range(n_graphs):
            m = bt == g
            mx_ref[g, :] = jnp.maximum(
                mx_ref[g, :], jnp.max(jnp.where(m, hp, neg), axis=0))
            mn_ref[g, :] = jnp.minimum(
                mn_ref[g, :], jnp.min(jnp.where(m, hp, pos), axis=0))

    return pl.pallas_call(
        body,
        grid=grid,
        in_specs=[
            pl.BlockSpec((te, f1), lambda i: (i, 0)),
            pl.BlockSpec((1, f1), lambda i: (0, 0)),
            pl.BlockSpec((f2, f1), lambda i: (0, 0)),
            pl.BlockSpec((1, f2), lambda i: (0, 0)),
            pl.BlockSpec((1, te, 1), lambda i: (i, 0, 0)),
        ],
        out_specs=[
            pl.BlockSpec((1, f2), lambda i: (0, 0)),
            pl.BlockSpec((1, f2), lambda i: (0, 0)),
            pl.BlockSpec((n_graphs, f2), lambda i: (0, 0)),
            pl.BlockSpec((n_graphs, f2), lambda i: (0, 0)),
        ],
        out_shape=[
            jax.ShapeDtypeStruct((1, f2), jnp.float32),
            jax.ShapeDtypeStruct((1, f2), jnp.float32),
            jax.ShapeDtypeStruct((n_graphs, f2), jnp.float32),
            jax.ShapeDtypeStruct((n_graphs, f2), jnp.float32),
        ],
    )(a, ba, Wb, bb, batch3)


def _finalize(s, ss, mx, mn, g, be, Wc1, bc1, Wc2, bc2, e_tot):
    n_graphs, f = mx.shape
    nc = Wc2.shape[0]
    fh = Wc1.shape[0]

    def body(s_ref, ss_ref, mx_ref, mn_ref, g_ref, be_ref,
             w1_ref, b1_ref, w2_ref, b2_ref, out_ref):
        mu = s_ref[...] / e_tot
        var = ss_ref[...] / e_tot - mu * mu
        gv = g_ref[...]
        scale = gv * lax.rsqrt(var + EPS)
        shift = be_ref[...] - mu * scale
        pooled_pre = jnp.where(gv >= 0.0, mx_ref[...], mn_ref[...])
        pooled = jnp.maximum(pooled_pre * scale + shift, 0.0)
        dn = (((1,), (1,)), ((), ()))
        z = jnp.maximum(
            lax.dot_general(pooled, w1_ref[...], dn,
                            preferred_element_type=jnp.float32, precision=lax.Precision.HIGHEST) + b1_ref[...],
            0.0)
        out_ref[...] = lax.dot_general(z, w2_ref[...], dn,
                                       preferred_element_type=jnp.float32, precision=lax.Precision.HIGHEST) \
            + b2_ref[...]

    return pl.pallas_call(
        body,
        out_shape=jax.ShapeDtypeStruct((n_graphs, nc), jnp.float32),
    )(s, ss, mx, mn, g, be, Wc1, bc1, Wc2, bc2)


def kernel(x, edge_index, batch,
           W1a, b1a, W1b, b1b, g1, be1,
           W2a, b2a, W2b, b2b, g2, be2,
           W3a, b3a, W3b, b3b, g3, be3,
           Wc1, bc1, Wc2, bc2):
    n = x.shape[0]
    e = edge_index.shape[1]
    e_tot = float(e)
    row = edge_index[0]
    col = edge_index[1]

    def split(Wa):
        f = Wa.shape[1] // 2
        return Wa[:, :f] - Wa[:, f:], Wa[:, f:]

    Wd1, Wr1 = split(W1a)
    Wd2, Wr2 = split(W2a)
    Wd3, Wr3 = split(W3a)
    r2 = lambda t: t.reshape(1, -1)

    u1, v1 = _uv_first(x, Wd1, Wr1, tn=2000)
    a1 = _sc_gather_add(u1, v1, row, col)
    hp1, s1, ss1 = _edge_mlp_stats(a1, r2(b1a), W1b, r2(b1b), n_keep=n, te=2000)

    u2, v2 = _norm_uv(hp1, s1, ss1, r2(g1), r2(be1), Wd2, Wr2, e_tot, tn=2000)
    a2 = _sc_gather_add(u2, v2, row, col)
    hp2, s2, ss2 = _edge_mlp_stats(a2, r2(b2a), W2b, r2(b2b), n_keep=n, te=2000)

    u3, v3 = _norm_uv(hp2, s2, ss2, r2(g2), r2(be2), Wd3, Wr3, e_tot, tn=2000)
    a3 = _sc_gather_add(u3, v3, row, col)
    te3 = 2000
    batch3 = batch.reshape(e // te3, te3, 1)
    s3, ss3, mx, mn = _edge_mlp_pool(a3, r2(b3a), W3b, r2(b3b), batch3,
                                     n_graphs=8, te=te3)

    return _finalize(s3, ss3, mx, mn, r2(g3), r2(be3),
                     Wc1, r2(bc1), Wc2, r2(bc2), e_tot)


# speedup vs baseline: 2.3193x; 1.3741x over previous
import functools

import jax
import jax.numpy as jnp
from jax import lax
from jax.experimental import pallas as pl
from jax.experimental.pallas import tpu as pltpu
from jax.experimental.pallas import tpu_sc as plsc

EPS = 1e-5
NC_SC = 2
NS_SC = 16
NW = NC_SC * NS_SC
CHUNK = 128


def _sc_gather_add(u, v, row, col):
    E = row.shape[0]
    F = u.shape[1]
    n_chunks = E // CHUNK
    mesh = plsc.VectorSubcoreMesh(
        core_axis_name="c", subcore_axis_name="s",
        num_cores=NC_SC, num_subcores=NS_SC)

    @functools.partial(
        pl.kernel,
        out_type=jax.ShapeDtypeStruct((E, F), jnp.float32),
        mesh=mesh,
        scratch_types=[
            pltpu.VMEM((CHUNK,), jnp.int32),
            pltpu.VMEM((CHUNK,), jnp.int32),
            pltpu.VMEM((CHUNK, F), jnp.float32),
            pltpu.VMEM((CHUNK, F), jnp.float32),
            pltpu.SemaphoreType.DMA,
        ],
        compiler_params=pltpu.CompilerParams(use_tc_tiling_on_sc=(F % 128 == 0)),
    )
    def k(u_hbm, v_hbm, row_hbm, col_hbm, a_hbm, idx_r, idx_c, buf_u, buf_v, sem):
        wid = lax.axis_index("s") * NC_SC + lax.axis_index("c")
        my_chunks = (n_chunks - wid + NW - 1) // NW

        def chunk_body(i, carry):
            base = (wid + i * NW) * CHUNK
            pltpu.sync_copy(row_hbm.at[pl.ds(base, CHUNK)], idx_r)
            pltpu.sync_copy(col_hbm.at[pl.ds(base, CHUNK)], idx_c)
            cu = pltpu.async_copy(u_hbm.at[idx_r], buf_u, sem)
            cv = pltpu.async_copy(v_hbm.at[idx_c], buf_v, sem)
            cu.wait()
            cv.wait()

            def add_row(e, c2):
                for j in range(F // 16):
                    sl = pl.ds(j * 16, 16)
                    buf_u[e, sl] = buf_u[e, sl] + buf_v[e, sl]
                return c2

            lax.fori_loop(0, CHUNK, add_row, 0)
            pltpu.sync_copy(buf_u, a_hbm.at[pl.ds(base, CHUNK)])
            return carry

        lax.fori_loop(0, my_chunks, chunk_body, 0)

    return k(u, v, row, col)


def _uv_first(x, Wd, Wr, tn):
    n, f_in = x.shape
    f_out = Wd.shape[0]

    def body(x_ref, wd_ref, wr_ref, u_ref, v_ref):
        xb = x_ref[...]
        dn = (((1,), (1,)), ((), ()))
        u_ref[...] = lax.dot_general(xb, wd_ref[...], dn,
                                     preferred_element_type=jnp.float32, precision=lax.Precision.HIGHEST)
        v_ref[...] = lax.dot_general(xb, wr_ref[...], dn,
                                     preferred_element_type=jnp.float32, precision=lax.Precision.HIGHEST)

    return pl.pallas_call(
        body,
        grid=(n // tn,),
        in_specs=[
            pl.BlockSpec((tn, f_in), lambda i: (i, 0)),
            pl.BlockSpec((f_out, f_in), lambda i: (0, 0)),
            pl.BlockSpec((f_out, f_in), lambda i: (0, 0)),
        ],
        out_specs=[
            pl.BlockSpec((tn, f_out), lambda i: (i, 0)),
            pl.BlockSpec((tn, f_out), lambda i: (i, 0)),
        ],
        out_shape=[
            jax.ShapeDtypeStruct((n, f_out), jnp.float32),
            jax.ShapeDtypeStruct((n, f_out), jnp.float32),
        ],
    )(x, Wd, Wr)


def _norm_uv(hp, sa, ssa, sb, ssb, g, be, Wd, Wr, e_tot, tn):
    n, f_in = hp.shape
    f_out = Wd.shape[0]
    grid = (n // tn,)

    def body(hp_ref, sa_ref, ssa_ref, sb_ref, ssb_ref, g_ref, be_ref,
             wd_ref, wr_ref, u_ref, v_ref):
        mu = (sa_ref[...] + sb_ref[...]) / e_tot
        var = (ssa_ref[...] + ssb_ref[...]) / e_tot - mu * mu
        scale = g_ref[...] * lax.rsqrt(var + EPS)
        shift = be_ref[...] - mu * scale
        h = jnp.maximum(hp_ref[...] * scale + shift, 0.0)
        dn = (((1,), (1,)), ((), ()))
        u_ref[...] = lax.dot_general(h, wd_ref[...], dn,
                                     preferred_element_type=jnp.float32, precision=lax.Precision.HIGHEST)
        v_ref[...] = lax.dot_general(h, wr_ref[...], dn,
                                     preferred_element_type=jnp.float32, precision=lax.Precision.HIGHEST)

    vec = lambda f: pl.BlockSpec((1, f), lambda i: (0, 0))
    return pl.pallas_call(
        body,
        grid=grid,
        in_specs=[
            pl.BlockSpec((tn, f_in), lambda i: (i, 0)),
            vec(f_in), vec(f_in), vec(f_in), vec(f_in), vec(f_in), vec(f_in),
            pl.BlockSpec((f_out, f_in), lambda i: (0, 0)),
            pl.BlockSpec((f_out, f_in), lambda i: (0, 0)),
        ],
        out_specs=[
            pl.BlockSpec((tn, f_out), lambda i: (i, 0)),
            pl.BlockSpec((tn, f_out), lambda i: (i, 0)),
        ],
        out_shape=[
            jax.ShapeDtypeStruct((n, f_out), jnp.float32),
            jax.ShapeDtypeStruct((n, f_out), jnp.float32),
        ],
    )(hp, sa, ssa, sb, ssb, g, be, Wd, Wr)


def _edge_mlp_stats(a, ba, Wb, bb, n_keep, te):
    e, f1 = a.shape
    f2 = Wb.shape[0]
    grid = (e // te,)
    keep_tiles = n_keep // te

    def body(*refs):
        if keep_tiles:
            a_ref, ba_ref, wb_ref, bb_ref, hp_ref, s_ref, ss_ref = refs
        else:
            a_ref, ba_ref, wb_ref, bb_ref, s_ref, ss_ref = refs
        i = pl.program_id(0)
        h = jnp.maximum(a_ref[...] + ba_ref[...], 0.0)
        hp = lax.dot_general(h, wb_ref[...], (((1,), (1,)), ((), ())),
                             preferred_element_type=jnp.float32,
                             precision=lax.Precision.HIGHEST) + bb_ref[...]

        if keep_tiles:
            @pl.when(i < keep_tiles)
            def _():
                hp_ref[...] = hp

        @pl.when(i == 0)
        def _():
            s_ref[...] = jnp.zeros_like(s_ref)
            ss_ref[...] = jnp.zeros_like(ss_ref)

        s_ref[...] += hp.sum(axis=0, keepdims=True)
        ss_ref[...] += (hp * hp).sum(axis=0, keepdims=True)

    hp_spec = [pl.BlockSpec((te, f2),
                            lambda i: (jnp.minimum(i, keep_tiles - 1), 0))]
    hp_shape = [jax.ShapeDtypeStruct((n_keep, f2), jnp.float32)]
    return pl.pallas_call(
        body,
        grid=grid,
        in_specs=[
            pl.BlockSpec((te, f1), lambda i: (i, 0)),
            pl.BlockSpec((1, f1), lambda i: (0, 0)),
            pl.BlockSpec((f2, f1), lambda i: (0, 0)),
            pl.BlockSpec((1, f2), lambda i: (0, 0)),
        ],
        out_specs=(hp_spec if keep_tiles else []) + [
            pl.BlockSpec((1, f2), lambda i: (0, 0)),
            pl.BlockSpec((1, f2), lambda i: (0, 0)),
        ],
        out_shape=(hp_shape if keep_tiles else []) + [
            jax.ShapeDtypeStruct((1, f2), jnp.float32),
            jax.ShapeDtypeStruct((1, f2), jnp.float32),
        ],
    )(a, ba, Wb, bb)


def _edge_mlp_pool(a, ba, Wb, bb, batch3, n_graphs, te):
    e, f1 = a.shape
    f2 = Wb.shape[0]
    grid = (e // te,)
    neg = -3.4e38
    pos = 3.4e38

    def body(a_ref, ba_ref, wb_ref, bb_ref, bt_ref,
             s_ref, ss_ref, mx_ref, mn_ref):
        i = pl.program_id(0)
        h = jnp.maximum(a_ref[...] + ba_ref[...], 0.0)
        hp = lax.dot_general(h, wb_ref[...], (((1,), (1,)), ((), ())),
                             preferred_element_type=jnp.float32, precision=lax.Precision.HIGHEST) + bb_ref[...]

        @pl.when(i == 0)
        def _():
            s_ref[...] = jnp.zeros_like(s_ref)
            ss_ref[...] = jnp.zeros_like(ss_ref)
            mx_ref[...] = jnp.full_like(mx_ref, neg)
            mn_ref[...] = jnp.full_like(mn_ref, pos)

        s_ref[...] += hp.sum(axis=0, keepdims=True)
        ss_ref[...] += (hp * hp).sum(axis=0, keepdims=True)

        b0 = bt_ref[0, 0, 0]
        bl = bt_ref[0, te - 1, 0]
        gids = lax.broadcasted_iota(jnp.int32, (n_graphs, 1), 0)

        @pl.when(b0 == bl)
        def _():
            tmx = jnp.max(hp, axis=0, keepdims=True)
            tmn = jnp.min(hp, axis=0, keepdims=True)
            sel = gids == b0
            mx_ref[...] = jnp.where(
                sel, jnp.maximum(mx_ref[...], tmx), mx_ref[...])
            mn_ref[...] = jnp.where(
                sel, jnp.minimum(mn_ref[...], tmn), mn_ref[...])

        @pl.when(b0 != bl)
        def _():
            bt = bt_ref[0]
            for g in range(n_graphs):
                m = bt == g
                mx_ref[g, :] = jnp.maximum(
                    mx_ref[g, :], jnp.max(jnp.where(m, hp, neg), axis=0))
                mn_ref[g, :] = jnp.minimum(
                    mn_ref[g, :], jnp.min(jnp.where(m, hp, pos), axis=0))

    return pl.pallas_call(
        body,
        grid=grid,
        in_specs=[
            pl.BlockSpec((te, f1), lambda i: (i, 0)),
            pl.BlockSpec((1, f1), lambda i: (0, 0)),
            pl.BlockSpec((f2, f1), lambda i: (0, 0)),
            pl.BlockSpec((1, f2), lambda i: (0, 0)),
            pl.BlockSpec((1, te, 1), lambda i: (i, 0, 0)),
        ],
        out_specs=[
            pl.BlockSpec((1, f2), lambda i: (0, 0)),
            pl.BlockSpec((1, f2), lambda i: (0, 0)),
            pl.BlockSpec((n_graphs, f2), lambda i: (0, 0)),
            pl.BlockSpec((n_graphs, f2), lambda i: (0, 0)),
        ],
        out_shape=[
            jax.ShapeDtypeStruct((1, f2), jnp.float32),
            jax.ShapeDtypeStruct((1, f2), jnp.float32),
            jax.ShapeDtypeStruct((n_graphs, f2), jnp.float32),
            jax.ShapeDtypeStruct((n_graphs, f2), jnp.float32),
        ],
    )(a, ba, Wb, bb, batch3)


def _finalize(sa, ssa, sb, ssb, mxa, mna, mxb, mnb,
              g, be, Wc1, bc1, Wc2, bc2, e_tot):
    n_graphs, f = mxa.shape
    nc = Wc2.shape[0]

    def body(sa_ref, ssa_ref, sb_ref, ssb_ref,
             mxa_ref, mna_ref, mxb_ref, mnb_ref, g_ref, be_ref,
             w1_ref, b1_ref, w2_ref, b2_ref, out_ref):
        mu = (sa_ref[...] + sb_ref[...]) / e_tot
        var = (ssa_ref[...] + ssb_ref[...]) / e_tot - mu * mu
        gv = g_ref[...]
        scale = gv * lax.rsqrt(var + EPS)
        shift = be_ref[...] - mu * scale
        mx = jnp.maximum(mxa_ref[...], mxb_ref[...])
        mn = jnp.minimum(mna_ref[...], mnb_ref[...])
        pooled_pre = jnp.where(gv >= 0.0, mx, mn)
        pooled = jnp.maximum(pooled_pre * scale + shift, 0.0)
        dn = (((1,), (1,)), ((), ()))
        z = jnp.maximum(
            lax.dot_general(pooled, w1_ref[...], dn,
                            preferred_element_type=jnp.float32, precision=lax.Precision.HIGHEST) + b1_ref[...],
            0.0)
        out_ref[...] = lax.dot_general(z, w2_ref[...], dn,
                                       preferred_element_type=jnp.float32, precision=lax.Precision.HIGHEST) \
            + b2_ref[...]

    return pl.pallas_call(
        body,
        out_shape=jax.ShapeDtypeStruct((n_graphs, nc), jnp.float32),
    )(sa, ssa, sb, ssb, mxa, mna, mxb, mnb, g, be, Wc1, bc1, Wc2, bc2)


def kernel(x, edge_index, batch,
           W1a, b1a, W1b, b1b, g1, be1,
           W2a, b2a, W2b, b2b, g2, be2,
           W3a, b3a, W3b, b3b, g3, be3,
           Wc1, bc1, Wc2, bc2):
    n = x.shape[0]
    e = edge_index.shape[1]
    e_tot = float(e)
    row = edge_index[0]
    col = edge_index[1]

    def split(Wa):
        f = Wa.shape[1] // 2
        return Wa[:, :f] - Wa[:, f:], Wa[:, f:]

    Wd1, Wr1 = split(W1a)
    Wd2, Wr2 = split(W2a)
    Wd3, Wr3 = split(W3a)
    r2 = lambda t: t.reshape(1, -1)

    e2 = e // 2
    row_a, col_a = row[:e2], col[:e2]
    row_b, col_b = row[e2:], col[e2:]
    te = 2000

    u1, v1 = _uv_first(x, Wd1, Wr1, tn=2000)
    a1a = _sc_gather_add(u1, v1, row_a, col_a)
    a1b = _sc_gather_add(u1, v1, row_b, col_b)
    hp1, s1a, ss1a = _edge_mlp_stats(a1a, r2(b1a), W1b, r2(b1b), n_keep=n, te=te)
    s1b, ss1b = _edge_mlp_stats(a1b, r2(b1a), W1b, r2(b1b), n_keep=0, te=te)

    u2, v2 = _norm_uv(hp1, s1a, ss1a, s1b, ss1b, r2(g1), r2(be1),
                      Wd2, Wr2, e_tot, tn=2000)
    a2a = _sc_gather_add(u2, v2, row_a, col_a)
    a2b = _sc_gather_add(u2, v2, row_b, col_b)
    hp2, s2a, ss2a = _edge_mlp_stats(a2a, r2(b2a), W2b, r2(b2b), n_keep=n, te=te)
    s2b, ss2b = _edge_mlp_stats(a2b, r2(b2a), W2b, r2(b2b), n_keep=0, te=te)

    u3, v3 = _norm_uv(hp2, s2a, ss2a, s2b, ss2b, r2(g2), r2(be2),
                      Wd3, Wr3, e_tot, tn=2000)
    a3a = _sc_gather_add(u3, v3, row_a, col_a)
    a3b = _sc_gather_add(u3, v3, row_b, col_b)
    batch3a = batch[:e2].reshape(e2 // te, te, 1)
    batch3b = batch[e2:].reshape(e2 // te, te, 1)
    s3a, ss3a, mxa, mna = _edge_mlp_pool(a3a, r2(b3a), W3b, r2(b3b), batch3a,
                                         n_graphs=8, te=te)
    s3b, ss3b, mxb, mnb = _edge_mlp_pool(a3b, r2(b3a), W3b, r2(b3b), batch3b,
                                         n_graphs=8, te=te)

    return _finalize(s3a, ss3a, s3b, ss3b, mxa, mna, mxb, mnb,
                     r2(g3), r2(be3), Wc1, r2(bc1), Wc2, r2(bc2), e_tot)
